# baseline (speedup 1.0000x reference)
import functools

import jax
import jax.numpy as jnp
from jax import lax
from jax.experimental import pallas as pl
from jax.experimental.pallas import tpu as pltpu
from jax.experimental.pallas import tpu_sc as plsc

N, E, D, H = 10000, 320000, 128, 64
H2 = H // 2
NC, NS = 2, 16
CH = 125
K = 80
NP = 10240
RT = NP // NS
RB = 1280
NBUF = 4

_sc_params = pltpu.CompilerParams(use_tc_tiling_on_sc=False)
_sc_mesh = plsc.VectorSubcoreMesh(
    core_axis_name="c", subcore_axis_name="s", num_cores=NC, num_subcores=NS
)


def _mm1_body(x_ref, wl_ref, wr_ref, b1_ref, yla_ref, ylb_ref, yr_ref):
    xb = x_ref[...]
    yl = jnp.dot(xb, wl_ref[...], preferred_element_type=jnp.float32)
    yla_ref[...] = yl[:, :H2]
    ylb_ref[...] = yl[:, H2:]
    yr_ref[...] = (
        jnp.dot(xb, wr_ref[...], preferred_element_type=jnp.float32) + b1_ref[...]
    )


_mm1 = pl.pallas_call(
    _mm1_body,
    grid=(NP // RB,),
    in_specs=[
        pl.BlockSpec((RB, D), lambda i: (i, 0)),
        pl.BlockSpec((D, H), lambda i: (0, 0)),
        pl.BlockSpec((D, H), lambda i: (0, 0)),
        pl.BlockSpec((1, H), lambda i: (0, 0)),
    ],
    out_specs=[
        pl.BlockSpec((RB, H2), lambda i: (i, 0)),
        pl.BlockSpec((RB, H2), lambda i: (i, 0)),
        pl.BlockSpec((RB, H), lambda i: (i, 0)),
    ],
    out_shape=[
        jax.ShapeDtypeStruct((NP, H2), jnp.float32),
        jax.ShapeDtypeStruct((NP, H2), jnp.float32),
        jax.ShapeDtypeStruct((NP, H), jnp.float32),
    ],
)


def _make_pass1(with_cnt):
    out_type = [jax.ShapeDtypeStruct((NC, NP, H2), jnp.float32)]
    scratch = [
        pltpu.VMEM((K, CH), jnp.int32),
        pltpu.VMEM((K, CH), jnp.int32),
        pltpu.VMEM((NBUF, CH, H2), jnp.float32),
        pltpu.VMEM_SHARED((NP, H2), jnp.float32),
        pltpu.VMEM_SHARED((NP, H2), jnp.float32),
        pltpu.SemaphoreType.DMA,
        pltpu.SemaphoreType.DMA,
    ]
    if with_cnt:
        out_type.append(jax.ShapeDtypeStruct((NC, NP), jnp.float32))
        scratch += [
            pltpu.VMEM((CH,), jnp.float32),
            pltpu.VMEM_SHARED((NP,), jnp.float32),
            pltpu.SemaphoreType.DMA,
        ]

    def body(ytab, srcj, dstj, zrows, zcnt, ones_in, *refs):
        if with_cnt:
            (out_s, out_c, src_v, dst_v, rows_v, ytab_sh, acc_sh,
             sem_g, sem_s, ones_v, cnt_sh, sem_c) = refs
        else:
            (out_s, src_v, dst_v, rows_v, ytab_sh, acc_sh,
             sem_g, sem_s) = refs
        c = lax.axis_index("c")
        s = lax.axis_index("s")
        pltpu.sync_copy(srcj.at[c, s], src_v)
        pltpu.sync_copy(dstj.at[c, s], dst_v)
        pltpu.sync_copy(ytab.at[pl.ds(s * RT, RT)], ytab_sh.at[pl.ds(s * RT, RT)])
        pltpu.sync_copy(zrows, acc_sh.at[pl.ds(s * RT, RT)])
        if with_cnt:
            pltpu.sync_copy(ones_in, ones_v)
            pltpu.sync_copy(zcnt, cnt_sh.at[pl.ds(s * RT, RT)])
        plsc.subcore_barrier()

        for b in range(NBUF):
            pltpu.async_copy(ytab_sh.at[src_v.at[b]], rows_v.at[b], sem_g)

        def step(j, carry):
            b = lax.rem(j, NBUF)
            pltpu.make_async_copy(
                ytab_sh.at[src_v.at[j]], rows_v.at[b], sem_g
            ).wait()
            pltpu.async_copy(rows_v.at[b], acc_sh.at[dst_v.at[j]], sem_s, add=True)
            if with_cnt:
                pltpu.async_copy(ones_v, cnt_sh.at[dst_v.at[j]], sem_c, add=True)
            nxt = j + NBUF

            @pl.when(nxt < K)
            def _():
                pltpu.make_async_copy(
                    rows_v.at[b], acc_sh.at[dst_v.at[j]], sem_s
                ).wait()
                pltpu.async_copy(ytab_sh.at[src_v.at[nxt]], rows_v.at[b], sem_g)

            return carry

        lax.fori_loop(0, K, step, 0)

        for b in range(NBUF):
            pltpu.make_async_copy(rows_v.at[b], acc_sh.at[dst_v.at[0]], sem_s).wait()
        if with_cnt:
            def drain(j, carry):
                pltpu.make_async_copy(ones_v, cnt_sh.at[dst_v.at[0]], sem_c).wait()
                return carry

            lax.fori_loop(0, K, drain, 0)
        plsc.subcore_barrier()
        pltpu.sync_copy(acc_sh.at[pl.ds(s * RT, RT)], out_s.at[c, pl.ds(s * RT, RT)])
        if with_cnt:
            pltpu.sync_copy(
                cnt_sh.at[pl.ds(s * RT, RT)], out_c.at[c, pl.ds(s * RT, RT)]
            )

    return pl.kernel(
        body,
        out_type=out_type,
        mesh=_sc_mesh,
        scratch_types=scratch,
        compiler_params=_sc_params,
    )


_sc_pass1a = _make_pass1(with_cnt=True)
_sc_pass1b = _make_pass1(with_cnt=False)


def _k2_body(s1a_ref, s1b_ref, cnt_ref, yr_ref, w2_ref, b2_ref, zr_ref):
    ssum = jnp.concatenate(
        [s1a_ref[0] + s1a_ref[1], s1b_ref[0] + s1b_ref[1]], axis=1
    )
    cnt = cnt_ref[0] + cnt_ref[1]
    agg = ssum / jnp.maximum(cnt, 1.0)
    h = jnp.maximum(agg + yr_ref[...], 0.0)
    zr_ref[...] = (
        jnp.dot(h, w2_ref[...], preferred_element_type=jnp.float32) + b2_ref[...]
    )


_k2 = pl.pallas_call(
    _k2_body,
    grid=(NP // RB,),
    in_specs=[
        pl.BlockSpec((NC, RB, H2), lambda i: (0, i, 0)),
        pl.BlockSpec((NC, RB, H2), lambda i: (0, i, 0)),
        pl.BlockSpec((NC, RB, 1), lambda i: (0, i, 0)),
        pl.BlockSpec((RB, H), lambda i: (i, 0)),
        pl.BlockSpec((H, 128), lambda i: (0, 0)),
        pl.BlockSpec((1, 128), lambda i: (0, 0)),
    ],
    out_specs=pl.BlockSpec((RB, 128), lambda i: (i, 0)),
    out_shape=jax.ShapeDtypeStruct((NP, 128), jnp.float32),
)


@functools.partial(
    pl.kernel,
    out_type=[jax.ShapeDtypeStruct((NC, NP), jnp.float32)],
    mesh=_sc_mesh,
    scratch_types=[
        pltpu.VMEM((K, CH), jnp.int32),
        pltpu.VMEM((K, CH), jnp.int32),
        pltpu.VMEM((NBUF, CH), jnp.float32),
        pltpu.VMEM_SHARED((NP,), jnp.float32),
        pltpu.VMEM_SHARED((NP,), jnp.float32),
        pltpu.SemaphoreType.DMA,
        pltpu.SemaphoreType.DMA,
    ],
    compiler_params=_sc_params,
)
def _sc_pass2(ztab, srcj, dstj, zcnt, out2, src_v, dst_v, vals_v, ztab_sh,
              acc_sh, sem_g, sem_s):
    c = lax.axis_index("c")
    s = lax.axis_index("s")
    pltpu.sync_copy(srcj.at[c, s], src_v)
    pltpu.sync_copy(dstj.at[c, s], dst_v)
    pltpu.sync_copy(ztab.at[pl.ds(s * RT, RT)], ztab_sh.at[pl.ds(s * RT, RT)])
    pltpu.sync_copy(zcnt, acc_sh.at[pl.ds(s * RT, RT)])
    plsc.subcore_barrier()

    for b in range(NBUF):
        pltpu.async_copy(ztab_sh.at[src_v.at[b]], vals_v.at[b], sem_g)

    def body(j, carry):
        b = lax.rem(j, NBUF)
        pltpu.make_async_copy(ztab_sh.at[src_v.at[j]], vals_v.at[b], sem_g).wait()
        pltpu.async_copy(vals_v.at[b], acc_sh.at[dst_v.at[j]], sem_s, add=True)
        nxt = j + NBUF

        @pl.when(nxt < K)
        def _():
            pltpu.make_async_copy(vals_v.at[b], acc_sh.at[dst_v.at[j]], sem_s).wait()
            pltpu.async_copy(ztab_sh.at[src_v.at[nxt]], vals_v.at[b], sem_g)

        return carry

    lax.fori_loop(0, K, body, 0)
    for b in range(NBUF):
        pltpu.make_async_copy(vals_v.at[b], acc_sh.at[dst_v.at[0]], sem_s).wait()
    plsc.subcore_barrier()
    pltpu.sync_copy(acc_sh.at[pl.ds(s * RT, RT)], out2.at[c, pl.ds(s * RT, RT)])


def _k3_body(s2_ref, cnt_ref, r2_ref, o_ref):
    s2 = s2_ref[0] + s2_ref[1]
    cnt = cnt_ref[0] + cnt_ref[1]
    o_ref[...] = s2 / jnp.maximum(cnt, 1.0) + r2_ref[...]


_k3 = pl.pallas_call(
    _k3_body,
    in_specs=[
        pl.BlockSpec((NC, NP // 128, 128), lambda: (0, 0, 0)),
        pl.BlockSpec((NC, NP // 128, 128), lambda: (0, 0, 0)),
        pl.BlockSpec((NP // 128, 128), lambda: (0, 0)),
    ],
    out_specs=pl.BlockSpec((NP // 128, 128), lambda: (0, 0)),
    out_shape=jax.ShapeDtypeStruct((NP // 128, 128), jnp.float32),
)


@jax.jit
def kernel(x, e, W1l, W1r, b1, W2l, W2r, b2):
    x_pad = jnp.pad(x, ((0, NP - N), (0, 0)))
    e5 = e.astype(jnp.int32).reshape(2, NC, NS, K, CH)
    srcj = e5[0]
    dstj = e5[1]

    zrows = jnp.zeros((RT, H2), jnp.float32)
    zcnt = jnp.zeros((RT,), jnp.float32)
    ones_in = jnp.ones((CH,), jnp.float32)

    yla, ylb, yr = _mm1(x_pad, W1l, W1r, b1.reshape(1, H))
    s1a, cntp = _sc_pass1a(yla, srcj, dstj, zrows, zcnt, ones_in)
    (s1b,) = _sc_pass1b(ylb, srcj, dstj, zrows, zcnt, ones_in)

    w2p = jnp.zeros((H, 128), jnp.float32)
    w2p = w2p.at[:, 0].set(W2l[:, 0]).at[:, 1].set(W2r[:, 0])
    bias2 = jnp.zeros((128,), jnp.float32).at[1].set(b2[0]).reshape(1, 128)
    zr = _k2(s1a, s1b, cntp.reshape(NC, NP, 1), yr, w2p, bias2)

    z = zr[:, 0]
    (s2p,) = _sc_pass2(z, srcj, dstj, zcnt)

    out = _k3(
        s2p.reshape(NC, NP // 128, 128),
        cntp.reshape(NC, NP // 128, 128),
        zr[:, 1].reshape(NP // 128, 128),
    )
    return out.reshape(NP, 1)[:N]


# speedup vs baseline: 1.0345x; 1.0345x over previous
import functools

import jax
import jax.numpy as jnp
from jax import lax
from jax.experimental import pallas as pl
from jax.experimental.pallas import tpu as pltpu
from jax.experimental.pallas import tpu_sc as plsc

N, E, D, H = 10000, 320000, 128, 64
H2 = H // 2
NC, NS = 2, 16
CH = 128
K = 80
EP = NC * NS * K * CH
NP = 10240
RT = NP // NS
RB = 1280
NBUF = 4

_sc_params = pltpu.CompilerParams(use_tc_tiling_on_sc=False)
_sc_mesh = plsc.VectorSubcoreMesh(
    core_axis_name="c", subcore_axis_name="s", num_cores=NC, num_subcores=NS
)


def _mm1_body(x_ref, wl_ref, wr_ref, b1_ref, yla_ref, ylb_ref, yr_ref):
    xb = x_ref[...]
    yl = jnp.dot(xb, wl_ref[...], preferred_element_type=jnp.float32)
    yla_ref[...] = yl[:, :H2]
    ylb_ref[...] = yl[:, H2:]
    yr_ref[...] = (
        jnp.dot(xb, wr_ref[...], preferred_element_type=jnp.float32) + b1_ref[...]
    )


_mm1 = pl.pallas_call(
    _mm1_body,
    grid=(NP // RB,),
    in_specs=[
        pl.BlockSpec((RB, D), lambda i: (i, 0)),
        pl.BlockSpec((D, H), lambda i: (0, 0)),
        pl.BlockSpec((D, H), lambda i: (0, 0)),
        pl.BlockSpec((1, H), lambda i: (0, 0)),
    ],
    out_specs=[
        pl.BlockSpec((RB, H2), lambda i: (i, 0)),
        pl.BlockSpec((RB, H2), lambda i: (i, 0)),
        pl.BlockSpec((RB, H), lambda i: (i, 0)),
    ],
    out_shape=[
        jax.ShapeDtypeStruct((NP, H2), jnp.float32),
        jax.ShapeDtypeStruct((NP, H2), jnp.float32),
        jax.ShapeDtypeStruct((NP, H), jnp.float32),
    ],
)


def _make_pass1(with_cnt):
    out_type = [jax.ShapeDtypeStruct((NC, NP, H2), jnp.float32)]
    scratch = [
        pltpu.VMEM((K, CH), jnp.int32),
        pltpu.VMEM((K, CH), jnp.int32),
        pltpu.VMEM((NBUF, CH, H2), jnp.float32),
        pltpu.VMEM_SHARED((NP, H2), jnp.float32),
        pltpu.VMEM_SHARED((NP, H2), jnp.float32),
        pltpu.SemaphoreType.DMA,
        pltpu.SemaphoreType.DMA,
    ]
    if with_cnt:
        out_type.append(jax.ShapeDtypeStruct((NC, NP), jnp.float32))
        scratch += [
            pltpu.VMEM((CH,), jnp.float32),
            pltpu.VMEM_SHARED((NP,), jnp.float32),
            pltpu.SemaphoreType.DMA,
        ]

    def body(ytab, srcj, dstj, zrows, zcnt, ones_in, *refs):
        if with_cnt:
            (out_s, out_c, src_v, dst_v, rows_v, ytab_sh, acc_sh,
             sem_g, sem_s, ones_v, cnt_sh, sem_c) = refs
        else:
            (out_s, src_v, dst_v, rows_v, ytab_sh, acc_sh,
             sem_g, sem_s) = refs
        c = lax.axis_index("c")
        s = lax.axis_index("s")
        pltpu.sync_copy(srcj.at[c, s], src_v)
        pltpu.sync_copy(dstj.at[c, s], dst_v)
        pltpu.sync_copy(ytab.at[pl.ds(s * RT, RT)], ytab_sh.at[pl.ds(s * RT, RT)])
        pltpu.sync_copy(zrows, acc_sh.at[pl.ds(s * RT, RT)])
        if with_cnt:
            pltpu.sync_copy(ones_in, ones_v)
            pltpu.sync_copy(zcnt, cnt_sh.at[pl.ds(s * RT, RT)])
        plsc.subcore_barrier()

        for b in range(NBUF):
            pltpu.async_copy(ytab_sh.at[src_v.at[b]], rows_v.at[b], sem_g)

        def step(j, carry):
            b = lax.rem(j, NBUF)
            pltpu.make_async_copy(
                ytab_sh.at[src_v.at[j]], rows_v.at[b], sem_g
            ).wait()
            pltpu.async_copy(rows_v.at[b], acc_sh.at[dst_v.at[j]], sem_s, add=True)
            if with_cnt:
                pltpu.async_copy(ones_v, cnt_sh.at[dst_v.at[j]], sem_c, add=True)
            nxt = j + NBUF

            @pl.when(nxt < K)
            def _():
                pltpu.make_async_copy(
                    rows_v.at[b], acc_sh.at[dst_v.at[j]], sem_s
                ).wait()
                pltpu.async_copy(ytab_sh.at[src_v.at[nxt]], rows_v.at[b], sem_g)

            return carry

        lax.fori_loop(0, K, step, 0)

        for b in range(NBUF):
            pltpu.make_async_copy(rows_v.at[b], acc_sh.at[dst_v.at[0]], sem_s).wait()
        if with_cnt:
            def drain(j, carry):
                pltpu.make_async_copy(ones_v, cnt_sh.at[dst_v.at[0]], sem_c).wait()
                return carry

            lax.fori_loop(0, K, drain, 0)
        plsc.subcore_barrier()
        pltpu.sync_copy(acc_sh.at[pl.ds(s * RT, RT)], out_s.at[c, pl.ds(s * RT, RT)])
        if with_cnt:
            pltpu.sync_copy(
                cnt_sh.at[pl.ds(s * RT, RT)], out_c.at[c, pl.ds(s * RT, RT)]
            )

    return pl.kernel(
        body,
        out_type=out_type,
        mesh=_sc_mesh,
        scratch_types=scratch,
        compiler_params=_sc_params,
    )


_sc_pass1a = _make_pass1(with_cnt=True)
_sc_pass1b = _make_pass1(with_cnt=False)


def _k2_body(s1a_ref, s1b_ref, cnt_ref, yr_ref, w2_ref, b2_ref, zr_ref):
    ssum = jnp.concatenate(
        [s1a_ref[0] + s1a_ref[1], s1b_ref[0] + s1b_ref[1]], axis=1
    )
    cnt = cnt_ref[0] + cnt_ref[1]
    agg = ssum / jnp.maximum(cnt, 1.0)
    h = jnp.maximum(agg + yr_ref[...], 0.0)
    zr_ref[...] = (
        jnp.dot(h, w2_ref[...], preferred_element_type=jnp.float32) + b2_ref[...]
    )


_k2 = pl.pallas_call(
    _k2_body,
    grid=(NP // RB,),
    in_specs=[
        pl.BlockSpec((NC, RB, H2), lambda i: (0, i, 0)),
        pl.BlockSpec((NC, RB, H2), lambda i: (0, i, 0)),
        pl.BlockSpec((NC, RB, 1), lambda i: (0, i, 0)),
        pl.BlockSpec((RB, H), lambda i: (i, 0)),
        pl.BlockSpec((H, 128), lambda i: (0, 0)),
        pl.BlockSpec((1, 128), lambda i: (0, 0)),
    ],
    out_specs=pl.BlockSpec((RB, 128), lambda i: (i, 0)),
    out_shape=jax.ShapeDtypeStruct((NP, 128), jnp.float32),
)


@functools.partial(
    pl.kernel,
    out_type=[jax.ShapeDtypeStruct((NC, NP), jnp.float32)],
    mesh=_sc_mesh,
    scratch_types=[
        pltpu.VMEM((K, CH), jnp.int32),
        pltpu.VMEM((K, CH), jnp.int32),
        pltpu.VMEM((NBUF, CH), jnp.float32),
        pltpu.VMEM_SHARED((NP,), jnp.float32),
        pltpu.VMEM_SHARED((NP,), jnp.float32),
        pltpu.SemaphoreType.DMA,
        pltpu.SemaphoreType.DMA,
    ],
    compiler_params=_sc_params,
)
def _sc_pass2(ztab, srcj, dstj, zcnt, out2, src_v, dst_v, vals_v, ztab_sh,
              acc_sh, sem_g, sem_s):
    c = lax.axis_index("c")
    s = lax.axis_index("s")
    pltpu.sync_copy(srcj.at[c, s], src_v)
    pltpu.sync_copy(dstj.at[c, s], dst_v)
    pltpu.sync_copy(ztab.at[pl.ds(s * RT, RT)], ztab_sh.at[pl.ds(s * RT, RT)])
    pltpu.sync_copy(zcnt, acc_sh.at[pl.ds(s * RT, RT)])
    plsc.subcore_barrier()

    for b in range(NBUF):
        pltpu.async_copy(ztab_sh.at[src_v.at[b]], vals_v.at[b], sem_g)

    def body(j, carry):
        b = lax.rem(j, NBUF)
        pltpu.make_async_copy(ztab_sh.at[src_v.at[j]], vals_v.at[b], sem_g).wait()
        pltpu.async_copy(vals_v.at[b], acc_sh.at[dst_v.at[j]], sem_s, add=True)
        nxt = j + NBUF

        @pl.when(nxt < K)
        def _():
            pltpu.make_async_copy(vals_v.at[b], acc_sh.at[dst_v.at[j]], sem_s).wait()
            pltpu.async_copy(ztab_sh.at[src_v.at[nxt]], vals_v.at[b], sem_g)

        return carry

    lax.fori_loop(0, K, body, 0)
    for b in range(NBUF):
        pltpu.make_async_copy(vals_v.at[b], acc_sh.at[dst_v.at[0]], sem_s).wait()
    plsc.subcore_barrier()
    pltpu.sync_copy(acc_sh.at[pl.ds(s * RT, RT)], out2.at[c, pl.ds(s * RT, RT)])


def _k3_body(s2_ref, cnt_ref, r2_ref, o_ref):
    s2 = s2_ref[0] + s2_ref[1]
    cnt = cnt_ref[0] + cnt_ref[1]
    o_ref[...] = s2 / jnp.maximum(cnt, 1.0) + r2_ref[...]


_k3 = pl.pallas_call(
    _k3_body,
    in_specs=[
        pl.BlockSpec((NC, NP // 128, 128), lambda: (0, 0, 0)),
        pl.BlockSpec((NC, NP // 128, 128), lambda: (0, 0, 0)),
        pl.BlockSpec((NP // 128, 128), lambda: (0, 0)),
    ],
    out_specs=pl.BlockSpec((NP // 128, 128), lambda: (0, 0)),
    out_shape=jax.ShapeDtypeStruct((NP // 128, 128), jnp.float32),
)


@jax.jit
def kernel(x, e, W1l, W1r, b1, W2l, W2r, b2):
    x_pad = jnp.pad(x, ((0, NP - N), (0, 0)))
    eint = e.astype(jnp.int32)
    pad_dst = N + jnp.arange(EP - E, dtype=jnp.int32) % (NP - N)
    src = jnp.concatenate([eint[0], jnp.zeros((EP - E,), jnp.int32)])
    dst = jnp.concatenate([eint[1], pad_dst])
    srcj = src.reshape(NC, NS, K, CH)
    dstj = dst.reshape(NC, NS, K, CH)

    zrows = jnp.zeros((RT, H2), jnp.float32)
    zcnt = jnp.zeros((RT,), jnp.float32)
    ones_in = jnp.ones((CH,), jnp.float32)

    yla, ylb, yr = _mm1(x_pad, W1l, W1r, b1.reshape(1, H))
    s1a, cntp = _sc_pass1a(yla, srcj, dstj, zrows, zcnt, ones_in)
    (s1b,) = _sc_pass1b(ylb, srcj, dstj, zrows, zcnt, ones_in)

    w2p = jnp.zeros((H, 128), jnp.float32)
    w2p = w2p.at[:, 0].set(W2l[:, 0]).at[:, 1].set(W2r[:, 0])
    bias2 = jnp.zeros((128,), jnp.float32).at[1].set(b2[0]).reshape(1, 128)
    zr = _k2(s1a, s1b, cntp.reshape(NC, NP, 1), yr, w2p, bias2)

    z = zr[:, 0]
    (s2p,) = _sc_pass2(z, srcj, dstj, zcnt)

    out = _k3(
        s2p.reshape(NC, NP // 128, 128),
        cntp.reshape(NC, NP // 128, 128),
        zr[:, 1].reshape(NP // 128, 128),
    )
    return out.reshape(NP, 1)[:N]


# speedup vs baseline: 1.1272x; 1.0896x over previous
import functools

import jax
import jax.numpy as jnp
from jax import lax
from jax.experimental import pallas as pl
from jax.experimental.pallas import tpu as pltpu
from jax.experimental.pallas import tpu_sc as plsc

N, E, D, H = 10000, 320000, 128, 64
H2 = H // 2
NC, NS = 2, 16
CH = 128
K = 80
EP = NC * NS * K * CH
NP = 10240
RT = NP // NS
RB = 1280
NBUF = 4

_sc_params = pltpu.CompilerParams(use_tc_tiling_on_sc=False)
_sc_mesh = plsc.VectorSubcoreMesh(
    core_axis_name="c", subcore_axis_name="s", num_cores=NC, num_subcores=NS
)


def _mm1_body(x_ref, wl_ref, wr_ref, b1_ref, yla_ref, ylb_ref, yr_ref):
    xb = x_ref[...]
    yl = jnp.dot(xb, wl_ref[...], preferred_element_type=jnp.float32)
    yla_ref[...] = yl[:, :H2]
    ylb_ref[...] = yl[:, H2:]
    yr_ref[...] = (
        jnp.dot(xb, wr_ref[...], preferred_element_type=jnp.float32) + b1_ref[...]
    )


_mm1 = pl.pallas_call(
    _mm1_body,
    grid=(NP // RB,),
    in_specs=[
        pl.BlockSpec((RB, D), lambda i: (i, 0)),
        pl.BlockSpec((D, H), lambda i: (0, 0)),
        pl.BlockSpec((D, H), lambda i: (0, 0)),
        pl.BlockSpec((1, H), lambda i: (0, 0)),
    ],
    out_specs=[
        pl.BlockSpec((RB, H2), lambda i: (i, 0)),
        pl.BlockSpec((RB, H2), lambda i: (i, 0)),
        pl.BlockSpec((RB, H), lambda i: (i, 0)),
    ],
    out_shape=[
        jax.ShapeDtypeStruct((NP, H2), jnp.float32),
        jax.ShapeDtypeStruct((NP, H2), jnp.float32),
        jax.ShapeDtypeStruct((NP, H), jnp.float32),
    ],
)


K2 = 2 * K


@functools.partial(
    pl.kernel,
    out_type=[
        jax.ShapeDtypeStruct((NC, NP, H2), jnp.float32),
        jax.ShapeDtypeStruct((NP,), jnp.float32),
    ],
    mesh=_sc_mesh,
    scratch_types=[
        pltpu.VMEM((K2, CH), jnp.int32),
        pltpu.VMEM((K2, CH), jnp.int32),
        pltpu.VMEM((NBUF, CH, H2), jnp.float32),
        pltpu.VMEM((CH,), jnp.float32),
        pltpu.VMEM_SHARED((NP, H2), jnp.float32),
        pltpu.VMEM_SHARED((NP, H2), jnp.float32),
        pltpu.VMEM_SHARED((NP,), jnp.float32),
        pltpu.SemaphoreType.DMA,
        pltpu.SemaphoreType.DMA,
        pltpu.SemaphoreType.DMA,
    ],
    compiler_params=_sc_params,
)
def _sc_pass1(yla, ylb, srcj, dstj, zrows, zcnt, ones_in, out_s, out_c,
              src_v, dst_v, rows_v, ones_v, ytab_sh, acc_sh, cnt_sh,
              sem_g, sem_s, sem_c):
    c = lax.axis_index("c")
    s = lax.axis_index("s")
    with_cnt = c == 0
    pltpu.sync_copy(srcj.at[s], src_v)
    pltpu.sync_copy(dstj.at[s], dst_v)
    pltpu.sync_copy(ones_in, ones_v)

    @pl.when(c == 0)
    def _():
        pltpu.sync_copy(yla.at[pl.ds(s * RT, RT)], ytab_sh.at[pl.ds(s * RT, RT)])

    @pl.when(c == 1)
    def _():
        pltpu.sync_copy(ylb.at[pl.ds(s * RT, RT)], ytab_sh.at[pl.ds(s * RT, RT)])

    pltpu.sync_copy(zrows, acc_sh.at[pl.ds(s * RT, RT)])
    pltpu.sync_copy(zcnt, cnt_sh.at[pl.ds(s * RT, RT)])
    plsc.subcore_barrier()

    for b in range(NBUF):
        pltpu.async_copy(ytab_sh.at[src_v.at[b]], rows_v.at[b], sem_g)

    def step(j, carry):
        b = lax.rem(j, NBUF)
        pltpu.make_async_copy(ytab_sh.at[src_v.at[j]], rows_v.at[b], sem_g).wait()
        pltpu.async_copy(rows_v.at[b], acc_sh.at[dst_v.at[j]], sem_s, add=True)

        @pl.when(with_cnt)
        def _():
            pltpu.async_copy(ones_v, cnt_sh.at[dst_v.at[j]], sem_c, add=True)

        nxt = j + NBUF

        @pl.when(nxt < K2)
        def _():
            pltpu.make_async_copy(rows_v.at[b], acc_sh.at[dst_v.at[j]], sem_s).wait()
            pltpu.async_copy(ytab_sh.at[src_v.at[nxt]], rows_v.at[b], sem_g)

        return carry

    lax.fori_loop(0, K2, step, 0)

    for b in range(NBUF):
        pltpu.make_async_copy(rows_v.at[b], acc_sh.at[dst_v.at[0]], sem_s).wait()

    @pl.when(with_cnt)
    def _():
        def drain(j, carry):
            pltpu.make_async_copy(ones_v, cnt_sh.at[dst_v.at[0]], sem_c).wait()
            return carry

        lax.fori_loop(0, K2, drain, 0)

    plsc.subcore_barrier()
    pltpu.sync_copy(acc_sh.at[pl.ds(s * RT, RT)], out_s.at[c, pl.ds(s * RT, RT)])

    @pl.when(with_cnt)
    def _():
        pltpu.sync_copy(cnt_sh.at[pl.ds(s * RT, RT)], out_c.at[pl.ds(s * RT, RT)])


def _k2_body(s1_ref, cnt_ref, yr_ref, w2_ref, b2_ref, zr_ref):
    ssum = jnp.concatenate([s1_ref[0], s1_ref[1]], axis=1)
    cnt = cnt_ref[...]
    agg = ssum / jnp.maximum(cnt, 1.0)
    h = jnp.maximum(agg + yr_ref[...], 0.0)
    zr_ref[...] = (
        jnp.dot(h, w2_ref[...], preferred_element_type=jnp.float32) + b2_ref[...]
    )


_k2 = pl.pallas_call(
    _k2_body,
    grid=(NP // RB,),
    in_specs=[
        pl.BlockSpec((NC, RB, H2), lambda i: (0, i, 0)),
        pl.BlockSpec((RB, 1), lambda i: (i, 0)),
        pl.BlockSpec((RB, H), lambda i: (i, 0)),
        pl.BlockSpec((H, 128), lambda i: (0, 0)),
        pl.BlockSpec((1, 128), lambda i: (0, 0)),
    ],
    out_specs=pl.BlockSpec((RB, 128), lambda i: (i, 0)),
    out_shape=jax.ShapeDtypeStruct((NP, 128), jnp.float32),
)


@functools.partial(
    pl.kernel,
    out_type=[jax.ShapeDtypeStruct((NC, NP), jnp.float32)],
    mesh=_sc_mesh,
    scratch_types=[
        pltpu.VMEM((K, CH), jnp.int32),
        pltpu.VMEM((K, CH), jnp.int32),
        pltpu.VMEM((NBUF, CH), jnp.float32),
        pltpu.VMEM_SHARED((NP,), jnp.float32),
        pltpu.VMEM_SHARED((NP,), jnp.float32),
        pltpu.SemaphoreType.DMA,
        pltpu.SemaphoreType.DMA,
    ],
    compiler_params=_sc_params,
)
def _sc_pass2(ztab, srcj, dstj, zcnt, out2, src_v, dst_v, vals_v, ztab_sh,
              acc_sh, sem_g, sem_s):
    c = lax.axis_index("c")
    s = lax.axis_index("s")
    pltpu.sync_copy(srcj.at[s, pl.ds(c * K, K)], src_v)
    pltpu.sync_copy(dstj.at[s, pl.ds(c * K, K)], dst_v)
    pltpu.sync_copy(ztab.at[pl.ds(s * RT, RT)], ztab_sh.at[pl.ds(s * RT, RT)])
    pltpu.sync_copy(zcnt, acc_sh.at[pl.ds(s * RT, RT)])
    plsc.subcore_barrier()

    for b in range(NBUF):
        pltpu.async_copy(ztab_sh.at[src_v.at[b]], vals_v.at[b], sem_g)

    def body(j, carry):
        b = lax.rem(j, NBUF)
        pltpu.make_async_copy(ztab_sh.at[src_v.at[j]], vals_v.at[b], sem_g).wait()
        pltpu.async_copy(vals_v.at[b], acc_sh.at[dst_v.at[j]], sem_s, add=True)
        nxt = j + NBUF

        @pl.when(nxt < K)
        def _():
            pltpu.make_async_copy(vals_v.at[b], acc_sh.at[dst_v.at[j]], sem_s).wait()
            pltpu.async_copy(ztab_sh.at[src_v.at[nxt]], vals_v.at[b], sem_g)

        return carry

    lax.fori_loop(0, K, body, 0)
    for b in range(NBUF):
        pltpu.make_async_copy(vals_v.at[b], acc_sh.at[dst_v.at[0]], sem_s).wait()
    plsc.subcore_barrier()
    pltpu.sync_copy(acc_sh.at[pl.ds(s * RT, RT)], out2.at[c, pl.ds(s * RT, RT)])


def _k3_body(s2_ref, cnt_ref, r2_ref, o_ref):
    s2 = s2_ref[0] + s2_ref[1]
    cnt = cnt_ref[...]
    o_ref[...] = s2 / jnp.maximum(cnt, 1.0) + r2_ref[...]


_k3 = pl.pallas_call(
    _k3_body,
    in_specs=[
        pl.BlockSpec((NC, NP // 128, 128), lambda: (0, 0, 0)),
        pl.BlockSpec((NP // 128, 128), lambda: (0, 0)),
        pl.BlockSpec((NP // 128, 128), lambda: (0, 0)),
    ],
    out_specs=pl.BlockSpec((NP // 128, 128), lambda: (0, 0)),
    out_shape=jax.ShapeDtypeStruct((NP // 128, 128), jnp.float32),
)


@jax.jit
def kernel(x, e, W1l, W1r, b1, W2l, W2r, b2):
    x_pad = jnp.pad(x, ((0, NP - N), (0, 0)))
    eint = e.astype(jnp.int32)
    pad_dst = N + jnp.arange(EP - E, dtype=jnp.int32) % (NP - N)
    src = jnp.concatenate([eint[0], jnp.zeros((EP - E,), jnp.int32)])
    dst = jnp.concatenate([eint[1], pad_dst])
    srcj = src.reshape(NS, 2 * K, CH)
    dstj = dst.reshape(NS, 2 * K, CH)

    zrows = jnp.zeros((RT, H2), jnp.float32)
    zcnt = jnp.zeros((RT,), jnp.float32)
    ones_in = jnp.ones((CH,), jnp.float32)

    yla, ylb, yr = _mm1(x_pad, W1l, W1r, b1.reshape(1, H))
    s1, cntp = _sc_pass1(yla, ylb, srcj, dstj, zrows, zcnt, ones_in)

    w2p = jnp.zeros((H, 128), jnp.float32)
    w2p = w2p.at[:, 0].set(W2l[:, 0]).at[:, 1].set(W2r[:, 0])
    bias2 = jnp.zeros((128,), jnp.float32).at[1].set(b2[0]).reshape(1, 128)
    zr = _k2(s1, cntp.reshape(NP, 1), yr, w2p, bias2)

    z = zr[:, 0]
    (s2p,) = _sc_pass2(z, srcj, dstj, zcnt)

    out = _k3(
        s2p.reshape(NC, NP // 128, 128),
        cntp.reshape(NP // 128, 128),
        zr[:, 1].reshape(NP // 128, 128),
    )
    return out.reshape(NP, 1)[:N]
